# baseline (speedup 1.0000x reference)
import jax
import jax.numpy as jnp
from jax.experimental import pallas as pl
from jax.experimental.pallas import tpu as pltpu


def _body(o_ref, s_ref, y_ref, out_ref, a1, a2, a3, a4):
    i = pl.program_id(0)
    nsteps = pl.num_programs(0)

    @pl.when(i == 0)
    def _init():
        a1[...] = jnp.zeros_like(a1)
        a2[...] = jnp.zeros_like(a2)
        a3[...] = jnp.zeros_like(a3)
        a4[...] = jnp.zeros_like(a4)

    o = o_ref[...]
    sv = s_ref[...]
    yv = y_ref[...]
    p = jax.nn.sigmoid(o)
    yf = yv.astype(jnp.float32)
    eq = yf == p
    pos = yv == 1
    prot = sv == 0

    one = jnp.float32(1.0)
    zero = jnp.float32(0.0)
    posf = jnp.where(pos, one, zero)
    pp = jnp.where(prot, posf, zero)
    pe = jnp.where(eq, posf, zero)
    pep = jnp.where(prot, pe, zero)

    a1[...] += jnp.sum(posf, axis=0)
    a2[...] += jnp.sum(pp, axis=0)
    a3[...] += jnp.sum(pe, axis=0)
    a4[...] += jnp.sum(pep, axis=0)

    @pl.when(i == nsteps - 1)
    def _epilogue():
        def _tot(a):
            r = jnp.sum(a[...], axis=0, keepdims=True)
            return jnp.sum(r, axis=1, keepdims=True)
        s1 = _tot(a1)
        s2 = _tot(a2)
        s3 = _tot(a3)
        s4 = _tot(a4)
        tp_p = s4
        den_p = s2
        tp_n = s3 - s4
        den_n = s1 - s2
        zero_v = jnp.zeros_like(s1)
        one_v = jnp.ones_like(s1)
        tpr_p = jnp.where(den_p == 0, zero_v, tp_p / jnp.maximum(den_p, one_v))
        tpr_n = jnp.where(den_n == 0, zero_v, tp_n / jnp.maximum(den_n, one_v))
        d = tpr_n - tpr_p
        g_pos = jnp.maximum(d, zero_v)
        g_neg = jnp.maximum(-d, zero_v)
        out_ref[...] = g_pos * g_pos + g_neg * g_neg + g_pos * g_pos + g_neg * g_neg


def kernel(X, out, sensitive, y):
    n = out.shape[0]
    tiles = n // 1024
    bt = min(512, tiles)
    o3 = out.reshape(tiles, 8, 128)
    s3 = sensitive.reshape(tiles, 8, 128)
    y3 = y.reshape(tiles, 8, 128)
    res = pl.pallas_call(
        _body,
        grid=(tiles // bt,),
        in_specs=[
            pl.BlockSpec((bt, 8, 128), lambda i: (i, 0, 0)),
            pl.BlockSpec((bt, 8, 128), lambda i: (i, 0, 0)),
            pl.BlockSpec((bt, 8, 128), lambda i: (i, 0, 0)),
        ],
        out_specs=pl.BlockSpec((1, 1), lambda i: (0, 0)),
        out_shape=jax.ShapeDtypeStruct((1, 1), jnp.float32),
        scratch_shapes=[
            pltpu.VMEM((8, 128), jnp.float32),
            pltpu.VMEM((8, 128), jnp.float32),
            pltpu.VMEM((8, 128), jnp.float32),
            pltpu.VMEM((8, 128), jnp.float32),
        ],
        compiler_params=pltpu.CompilerParams(
            dimension_semantics=("arbitrary",),
        ),
        name="ate_loss",
    )(o3, s3, y3)
    return res.reshape(())


# speedup vs baseline: 1.1006x; 1.1006x over previous
import jax
import jax.numpy as jnp
from jax.experimental import pallas as pl
from jax.experimental.pallas import tpu as pltpu

_BT = 512
_NBUF = 4


def _partials(o, sv, yv):
    p = jax.nn.sigmoid(o)
    yf = yv.astype(jnp.float32)
    eq = yf == p
    pos = yv == 1
    prot = sv == 0
    one = jnp.float32(1.0)
    zero = jnp.float32(0.0)
    posf = jnp.where(pos, one, zero)
    pp = jnp.where(prot, posf, zero)
    pe = jnp.where(eq, posf, zero)
    pep = jnp.where(prot, pe, zero)
    return (jnp.sum(posf, axis=0), jnp.sum(pp, axis=0),
            jnp.sum(pe, axis=0), jnp.sum(pep, axis=0))


def _make_body(nsteps):
    def _body(o_hbm, s_hbm, y_hbm, out_ref, ob, sb, yb, sem_o, sem_s, sem_y):
        def copies(k):
            slot = k % _NBUF
            sl = pl.ds(k * _BT, _BT)
            return (
                pltpu.make_async_copy(o_hbm.at[sl], ob.at[slot], sem_o.at[slot]),
                pltpu.make_async_copy(s_hbm.at[sl], sb.at[slot], sem_s.at[slot]),
                pltpu.make_async_copy(y_hbm.at[sl], yb.at[slot], sem_y.at[slot]),
            )

        for k in range(min(_NBUF, nsteps)):
            for c in copies(k):
                c.start()

        z = jnp.zeros((8, 128), jnp.float32)
        a1, a2, a3, a4 = z, z, z, z
        for k in range(nsteps):
            slot = k % _NBUF
            for c in copies(k):
                c.wait()
            d1, d2, d3, d4 = _partials(ob[slot], sb[slot], yb[slot])
            a1 = a1 + d1
            a2 = a2 + d2
            a3 = a3 + d3
            a4 = a4 + d4
            nxt = k + _NBUF
            if nxt < nsteps:
                for c in copies(nxt):
                    c.start()

        def _tot(a):
            r = jnp.sum(a, axis=0, keepdims=True)
            return jnp.sum(r, axis=1, keepdims=True)
        s1, s2, s3, s4 = _tot(a1), _tot(a2), _tot(a3), _tot(a4)
        tp_p = s4
        den_p = s2
        tp_n = s3 - s4
        den_n = s1 - s2
        zero_v = jnp.zeros_like(s1)
        one_v = jnp.ones_like(s1)
        tpr_p = jnp.where(den_p == 0, zero_v, tp_p / jnp.maximum(den_p, one_v))
        tpr_n = jnp.where(den_n == 0, zero_v, tp_n / jnp.maximum(den_n, one_v))
        d = tpr_n - tpr_p
        g_pos = jnp.maximum(d, zero_v)
        g_neg = jnp.maximum(-d, zero_v)
        out_ref[...] = g_pos * g_pos + g_neg * g_neg + g_pos * g_pos + g_neg * g_neg

    return _body


def kernel(X, out, sensitive, y):
    n = out.shape[0]
    tiles = n // 1024
    bt = min(_BT, tiles)
    nsteps = tiles // bt
    o3 = out.reshape(tiles, 8, 128)
    s3 = sensitive.reshape(tiles, 8, 128)
    y3 = y.reshape(tiles, 8, 128)
    res = pl.pallas_call(
        _make_body(nsteps),
        in_specs=[
            pl.BlockSpec(memory_space=pl.ANY),
            pl.BlockSpec(memory_space=pl.ANY),
            pl.BlockSpec(memory_space=pl.ANY),
        ],
        out_specs=pl.BlockSpec(memory_space=pltpu.VMEM),
        out_shape=jax.ShapeDtypeStruct((1, 1), jnp.float32),
        scratch_shapes=[
            pltpu.VMEM((_NBUF, bt, 8, 128), jnp.float32),
            pltpu.VMEM((_NBUF, bt, 8, 128), jnp.int32),
            pltpu.VMEM((_NBUF, bt, 8, 128), jnp.int32),
            pltpu.SemaphoreType.DMA((_NBUF,)),
            pltpu.SemaphoreType.DMA((_NBUF,)),
            pltpu.SemaphoreType.DMA((_NBUF,)),
        ],
        compiler_params=pltpu.CompilerParams(
            vmem_limit_bytes=56 * 1024 * 1024,
        ),
        name="ate_loss",
    )(o3, s3, y3)
    return res.reshape(())


# speedup vs baseline: 1.1368x; 1.0329x over previous
import jax
import jax.numpy as jnp
from jax.experimental import pallas as pl
from jax.experimental.pallas import tpu as pltpu

_BT = 256
_NBUF = 8


def _partials(o, sv, yv):
    p = jax.nn.sigmoid(o)
    yf = yv.astype(jnp.float32)
    eq = yf == p
    pos = yv == 1
    prot = sv == 0
    one = jnp.float32(1.0)
    zero = jnp.float32(0.0)
    posf = jnp.where(pos, one, zero)
    pp = jnp.where(prot, posf, zero)
    pe = jnp.where(eq, posf, zero)
    pep = jnp.where(prot, pe, zero)
    return (jnp.sum(posf, axis=0), jnp.sum(pp, axis=0),
            jnp.sum(pe, axis=0), jnp.sum(pep, axis=0))


def _make_body(nsteps):
    def _body(o_hbm, s_hbm, y_hbm, out_ref, ob, sb, yb, sem_o, sem_s, sem_y):
        def copies(k):
            slot = k % _NBUF
            sl = pl.ds(k * _BT, _BT)
            return (
                pltpu.make_async_copy(o_hbm.at[sl], ob.at[slot], sem_o.at[slot]),
                pltpu.make_async_copy(s_hbm.at[sl], sb.at[slot], sem_s.at[slot]),
                pltpu.make_async_copy(y_hbm.at[sl], yb.at[slot], sem_y.at[slot]),
            )

        for k in range(min(_NBUF, nsteps)):
            for c in copies(k):
                c.start()

        z = jnp.zeros((8, 128), jnp.float32)
        a1, a2, a3, a4 = z, z, z, z
        for k in range(nsteps):
            slot = k % _NBUF
            for c in copies(k):
                c.wait()
            d1, d2, d3, d4 = _partials(ob[slot], sb[slot], yb[slot])
            a1 = a1 + d1
            a2 = a2 + d2
            a3 = a3 + d3
            a4 = a4 + d4
            nxt = k + _NBUF
            if nxt < nsteps:
                for c in copies(nxt):
                    c.start()

        def _tot(a):
            r = jnp.sum(a, axis=0, keepdims=True)
            return jnp.sum(r, axis=1, keepdims=True)
        s1, s2, s3, s4 = _tot(a1), _tot(a2), _tot(a3), _tot(a4)
        tp_p = s4
        den_p = s2
        tp_n = s3 - s4
        den_n = s1 - s2
        zero_v = jnp.zeros_like(s1)
        one_v = jnp.ones_like(s1)
        tpr_p = jnp.where(den_p == 0, zero_v, tp_p / jnp.maximum(den_p, one_v))
        tpr_n = jnp.where(den_n == 0, zero_v, tp_n / jnp.maximum(den_n, one_v))
        d = tpr_n - tpr_p
        g_pos = jnp.maximum(d, zero_v)
        g_neg = jnp.maximum(-d, zero_v)
        out_ref[...] = g_pos * g_pos + g_neg * g_neg + g_pos * g_pos + g_neg * g_neg

    return _body


def kernel(X, out, sensitive, y):
    n = out.shape[0]
    tiles = n // 1024
    bt = min(_BT, tiles)
    nsteps = tiles // bt
    o3 = out.reshape(tiles, 8, 128)
    s3 = sensitive.reshape(tiles, 8, 128)
    y3 = y.reshape(tiles, 8, 128)
    res = pl.pallas_call(
        _make_body(nsteps),
        in_specs=[
            pl.BlockSpec(memory_space=pl.ANY),
            pl.BlockSpec(memory_space=pl.ANY),
            pl.BlockSpec(memory_space=pl.ANY),
        ],
        out_specs=pl.BlockSpec(memory_space=pltpu.VMEM),
        out_shape=jax.ShapeDtypeStruct((1, 1), jnp.float32),
        scratch_shapes=[
            pltpu.VMEM((_NBUF, bt, 8, 128), jnp.float32),
            pltpu.VMEM((_NBUF, bt, 8, 128), jnp.int32),
            pltpu.VMEM((_NBUF, bt, 8, 128), jnp.int32),
            pltpu.SemaphoreType.DMA((_NBUF,)),
            pltpu.SemaphoreType.DMA((_NBUF,)),
            pltpu.SemaphoreType.DMA((_NBUF,)),
        ],
        compiler_params=pltpu.CompilerParams(
            vmem_limit_bytes=56 * 1024 * 1024,
        ),
        name="ate_loss",
    )(o3, s3, y3)
    return res.reshape(())


# speedup vs baseline: 1.1790x; 1.0371x over previous
import jax
import jax.numpy as jnp
from jax.experimental import pallas as pl
from jax.experimental.pallas import tpu as pltpu

_BT = 128
_NBUF = 16


def _partials(o, sv, yv):
    p = jax.nn.sigmoid(o)
    yf = yv.astype(jnp.float32)
    eq = yf == p
    pos = yv == 1
    prot = sv == 0
    one = jnp.float32(1.0)
    zero = jnp.float32(0.0)
    posf = jnp.where(pos, one, zero)
    pp = jnp.where(prot, posf, zero)
    pe = jnp.where(eq, posf, zero)
    pep = jnp.where(prot, pe, zero)
    return (jnp.sum(posf, axis=0), jnp.sum(pp, axis=0),
            jnp.sum(pe, axis=0), jnp.sum(pep, axis=0))


def _make_body(nsteps):
    def _body(o_hbm, s_hbm, y_hbm, out_ref, ob, sb, yb, sem_o, sem_s, sem_y):
        def copies(k):
            slot = k % _NBUF
            sl = pl.ds(k * _BT, _BT)
            return (
                pltpu.make_async_copy(o_hbm.at[sl], ob.at[slot], sem_o.at[slot]),
                pltpu.make_async_copy(s_hbm.at[sl], sb.at[slot], sem_s.at[slot]),
                pltpu.make_async_copy(y_hbm.at[sl], yb.at[slot], sem_y.at[slot]),
            )

        for k in range(min(_NBUF, nsteps)):
            for c in copies(k):
                c.start()

        z = jnp.zeros((8, 128), jnp.float32)
        a1, a2, a3, a4 = z, z, z, z
        for k in range(nsteps):
            slot = k % _NBUF
            for c in copies(k):
                c.wait()
            d1, d2, d3, d4 = _partials(ob[slot], sb[slot], yb[slot])
            a1 = a1 + d1
            a2 = a2 + d2
            a3 = a3 + d3
            a4 = a4 + d4
            nxt = k + _NBUF
            if nxt < nsteps:
                for c in copies(nxt):
                    c.start()

        def _tot(a):
            r = jnp.sum(a, axis=0, keepdims=True)
            return jnp.sum(r, axis=1, keepdims=True)
        s1, s2, s3, s4 = _tot(a1), _tot(a2), _tot(a3), _tot(a4)
        tp_p = s4
        den_p = s2
        tp_n = s3 - s4
        den_n = s1 - s2
        zero_v = jnp.zeros_like(s1)
        one_v = jnp.ones_like(s1)
        tpr_p = jnp.where(den_p == 0, zero_v, tp_p / jnp.maximum(den_p, one_v))
        tpr_n = jnp.where(den_n == 0, zero_v, tp_n / jnp.maximum(den_n, one_v))
        d = tpr_n - tpr_p
        g_pos = jnp.maximum(d, zero_v)
        g_neg = jnp.maximum(-d, zero_v)
        out_ref[...] = g_pos * g_pos + g_neg * g_neg + g_pos * g_pos + g_neg * g_neg

    return _body


def kernel(X, out, sensitive, y):
    n = out.shape[0]
    tiles = n // 1024
    bt = min(_BT, tiles)
    nsteps = tiles // bt
    o3 = out.reshape(tiles, 8, 128)
    s3 = sensitive.reshape(tiles, 8, 128)
    y3 = y.reshape(tiles, 8, 128)
    res = pl.pallas_call(
        _make_body(nsteps),
        in_specs=[
            pl.BlockSpec(memory_space=pl.ANY),
            pl.BlockSpec(memory_space=pl.ANY),
            pl.BlockSpec(memory_space=pl.ANY),
        ],
        out_specs=pl.BlockSpec(memory_space=pltpu.VMEM),
        out_shape=jax.ShapeDtypeStruct((1, 1), jnp.float32),
        scratch_shapes=[
            pltpu.VMEM((_NBUF, bt, 8, 128), jnp.float32),
            pltpu.VMEM((_NBUF, bt, 8, 128), jnp.int32),
            pltpu.VMEM((_NBUF, bt, 8, 128), jnp.int32),
            pltpu.SemaphoreType.DMA((_NBUF,)),
            pltpu.SemaphoreType.DMA((_NBUF,)),
            pltpu.SemaphoreType.DMA((_NBUF,)),
        ],
        compiler_params=pltpu.CompilerParams(
            vmem_limit_bytes=56 * 1024 * 1024,
        ),
        name="ate_loss",
    )(o3, s3, y3)
    return res.reshape(())
